# baseline (speedup 1.0000x reference)
import functools

import jax
import jax.numpy as jnp
from jax import lax
from jax.experimental import pallas as pl
from jax.experimental.pallas import tpu as pltpu
from jax.experimental.pallas import tpu_sc as plsc

_N = 20000
_HW = 256
_W = 16
_R = 2000
_NW = 32
_NPAD = 20480
_EPW = _NPAD * 4 // _NW
_CH = 128
_NCH = _EPW // _CH


def _tc_body(cls_ref, rois_ref, scores_ref, mask_ref, fidx_ref, base_ref, scale_ref):
    i = pl.program_id(0)
    x = cls_ref[...]
    m = jnp.max(x, axis=1, keepdims=True)
    cell = lax.broadcasted_iota(jnp.int32, x.shape, 1)
    idx = jnp.min(jnp.where(x == m, cell, _HW), axis=1, keepdims=True)
    fw = (idx % _W).astype(jnp.float32)
    fh = (idx // _W).astype(jnp.float32)

    r = rois_ref[...]
    x1, y1, x2, y2 = r[:, 0:1], r[:, 1:2], r[:, 2:3], r[:, 3:4]
    cx = (x1 + x2) * 0.5
    cy = (y1 + y2) * 0.5
    hw = (x2 - x1) * 0.5
    hh = (y2 - y1) * 0.5
    nx1 = cx - hw
    ny1 = cy - hh
    bw = ((cx + hw) - nx1) * (1.0 / _W)
    bh = ((cy + hh) - ny1) * (1.0 / _W)

    scores_ref[...] = m
    mask_ref[...] = jnp.where(m >= 0.0, 1.0, 0.0)
    bx = bw * (fw + 0.5) + nx1
    by = bh * (fh + 0.5) + ny1
    base_ref[...] = jnp.concatenate([bx, by, bx, by], axis=1)
    scale_ref[...] = jnp.concatenate([bw, bh, bw, bh], axis=1)
    row = lax.broadcasted_iota(jnp.int32, (_R, 1), 0) + i * _R
    f0 = row * (4 * _HW) + idx
    fidx_ref[...] = jnp.concatenate(
        [f0, f0 + _HW, f0 + 2 * _HW, f0 + 3 * _HW], axis=1)


def _tc_stage(cls2d, rois):
    grid = _N // _R
    return pl.pallas_call(
        _tc_body,
        grid=(grid,),
        in_specs=[
            pl.BlockSpec((_R, _HW), lambda i: (i, 0)),
            pl.BlockSpec((_R, 4), lambda i: (i, 0)),
        ],
        out_specs=[
            pl.BlockSpec((_R, 1), lambda i: (i, 0)),
            pl.BlockSpec((_R, 1), lambda i: (i, 0)),
            pl.BlockSpec((_R, 4), lambda i: (i, 0)),
            pl.BlockSpec((_R, 4), lambda i: (i, 0)),
            pl.BlockSpec((_R, 4), lambda i: (i, 0)),
        ],
        out_shape=[
            jax.ShapeDtypeStruct((_N, 1), jnp.float32),
            jax.ShapeDtypeStruct((_N, 1), jnp.float32),
            jax.ShapeDtypeStruct((_N, 4), jnp.int32),
            jax.ShapeDtypeStruct((_N, 4), jnp.float32),
            jax.ShapeDtypeStruct((_N, 4), jnp.float32),
        ],
    )(cls2d, rois)


def _sc_gather_body(reg_hbm, fidx_hbm, base_hbm, scale_hbm, out_hbm,
                    idx_v, off_v, base_v, scale_v, out_v, sem):
    wid = lax.axis_index("s") * 2 + lax.axis_index("c")
    pltpu.sync_copy(fidx_hbm.at[wid], idx_v)
    pltpu.sync_copy(base_hbm.at[wid], base_v)
    pltpu.sync_copy(scale_hbm.at[wid], scale_v)
    copies = []
    for j in range(_NCH):
        copies.append(pltpu.async_copy(
            reg_hbm.at[idx_v.at[j]], off_v.at[pl.ds(j * _CH, _CH)], sem))
    for c in copies:
        c.wait()

    def body(t, carry):
        sl = pl.ds(t * 16, 16)
        out_v[sl] = base_v[sl] + scale_v[sl] * off_v[sl]
        return carry

    lax.fori_loop(0, _EPW // 16, body, 0)
    pltpu.sync_copy(out_v, out_hbm.at[wid])


def _sc_stage(reg_flat, fidx3d, base2d, scale2d):
    mesh = plsc.VectorSubcoreMesh(core_axis_name="c", subcore_axis_name="s")
    fn = functools.partial(
        pl.kernel,
        mesh=mesh,
        out_type=jax.ShapeDtypeStruct((_NW, _EPW), jnp.float32),
        scratch_types=[
            pltpu.VMEM((_NCH, _CH), jnp.int32),
            pltpu.VMEM((_EPW,), jnp.float32),
            pltpu.VMEM((_EPW,), jnp.float32),
            pltpu.VMEM((_EPW,), jnp.float32),
            pltpu.VMEM((_EPW,), jnp.float32),
            pltpu.SemaphoreType.DMA,
        ],
    )(_sc_gather_body)
    return fn(reg_flat, fidx3d, base2d, scale2d)


def kernel(batch_rois, rcnn_cls_pred, rcnn_reg_pred):
    batch = batch_rois.shape[0]
    cls2d = rcnn_cls_pred.reshape(_N, _HW)
    rois = batch_rois.reshape(_N, 4)
    scores, maskf, fidx, base, scale = _tc_stage(cls2d, rois)

    pad = ((0, _NPAD - _N), (0, 0))
    fidx3d = jnp.pad(fidx, pad).reshape(_NW, _NCH, _CH)
    base2d = jnp.pad(base, pad).reshape(_NW, _EPW)
    scale2d = jnp.pad(scale, pad).reshape(_NW, _EPW)
    reg_flat = rcnn_reg_pred.reshape(-1)

    boxes_flat = _sc_stage(reg_flat, fidx3d, base2d, scale2d)
    boxes = boxes_flat.reshape(_NPAD, 4)[:_N].reshape(batch, -1, 4)
    scores_out = scores.reshape(batch, -1, 1)
    labels = jnp.zeros_like(scores_out)
    keep_mask = maskf.astype(jnp.bool_).reshape(batch, -1, 1)
    return boxes, scores_out, labels, keep_mask


# speedup vs baseline: 2.0354x; 2.0354x over previous
import jax
import jax.numpy as jnp
from jax import lax
from jax.experimental import pallas as pl

_N = 20000
_HW = 256
_W = 16
_B = 8
_C = 2500


def _stage1_body(cls_ref, rois_ref, scores_ref, mask_ref, idx_ref,
                 base_ref, scale_ref):
    x = cls_ref[...]
    m = jnp.max(x, axis=0)
    cell = lax.broadcasted_iota(jnp.int32, x.shape, 0)
    idx = jnp.min(jnp.where(x == m[None], cell, _HW), axis=0)
    fw = (idx % _W).astype(jnp.float32)
    fh = (idx // _W).astype(jnp.float32)

    r = rois_ref[...]
    x1, y1, x2, y2 = r[:, 0, :], r[:, 1, :], r[:, 2, :], r[:, 3, :]
    cx = (x1 + x2) * 0.5
    cy = (y1 + y2) * 0.5
    hw = (x2 - x1) * 0.5
    hh = (y2 - y1) * 0.5
    nx1 = cx - hw
    ny1 = cy - hh
    bw = ((cx + hw) - nx1) * (1.0 / _W)
    bh = ((cy + hh) - ny1) * (1.0 / _W)

    scores_ref[...] = m
    mask_ref[...] = jnp.where(m >= 0.0, 1.0, 0.0)
    idx_ref[...] = idx
    bx = bw * (fw + 0.5) + nx1
    by = bh * (fh + 0.5) + ny1
    base_ref[0] = bx
    base_ref[1] = by
    base_ref[2] = bx
    base_ref[3] = by
    scale_ref[0] = bw
    scale_ref[1] = bh
    scale_ref[2] = bw
    scale_ref[3] = bh


def _stage1(cls3, rois3):
    return pl.pallas_call(
        _stage1_body,
        in_specs=[
            pl.BlockSpec((_HW, _B, _C), lambda: (0, 0, 0)),
            pl.BlockSpec((_B, 4, _C), lambda: (0, 0, 0)),
        ],
        out_specs=[
            pl.BlockSpec((_B, _C), lambda: (0, 0)),
            pl.BlockSpec((_B, _C), lambda: (0, 0)),
            pl.BlockSpec((_B, _C), lambda: (0, 0)),
            pl.BlockSpec((4, _B, _C), lambda: (0, 0, 0)),
            pl.BlockSpec((4, _B, _C), lambda: (0, 0, 0)),
        ],
        out_shape=[
            jax.ShapeDtypeStruct((_B, _C), jnp.float32),
            jax.ShapeDtypeStruct((_B, _C), jnp.float32),
            jax.ShapeDtypeStruct((_B, _C), jnp.int32),
            jax.ShapeDtypeStruct((4, _B, _C), jnp.float32),
            jax.ShapeDtypeStruct((4, _B, _C), jnp.float32),
        ],
    )(cls3, rois3)


def _stage2_body(reg_ref, idx_ref, base_ref, scale_ref, out_ref):
    rg = reg_ref[...]
    idx = idx_ref[...]
    cell = lax.broadcasted_iota(jnp.int32, rg.shape, 0)
    off = jnp.sum(jnp.where(cell == idx[None], rg, 0.0), axis=0)
    out_ref[...] = base_ref[...] + scale_ref[...] * off[None]


def _stage2(reg3, idx8, base4, scale4):
    return pl.pallas_call(
        _stage2_body,
        grid=(4,),
        in_specs=[
            pl.BlockSpec((_HW, _B, _C), lambda g: (g, 0, 0)),
            pl.BlockSpec((_B, _C), lambda g: (0, 0)),
            pl.BlockSpec((1, _B, _C), lambda g: (g, 0, 0)),
            pl.BlockSpec((1, _B, _C), lambda g: (g, 0, 0)),
        ],
        out_specs=pl.BlockSpec((1, _B, _C), lambda g: (g, 0, 0)),
        out_shape=jax.ShapeDtypeStruct((4, _B, _C), jnp.float32),
    )(reg3, idx8, base4, scale4)


def kernel(batch_rois, rcnn_cls_pred, rcnn_reg_pred):
    cls3 = rcnn_cls_pred.reshape(_N, _HW).T.reshape(_HW, _B, _C)
    reg3 = rcnn_reg_pred.reshape(_N, 4 * _HW).T.reshape(4 * _HW, _B, _C)
    rois3 = jnp.transpose(batch_rois, (0, 2, 1))

    scores8, mask8, idx8, base4, scale4 = _stage1(cls3, rois3)
    boxes4 = _stage2(reg3, idx8, base4, scale4)

    boxes = jnp.transpose(boxes4, (1, 2, 0))
    scores = scores8[..., None]
    labels = jnp.zeros_like(scores)
    keep_mask = mask8.astype(jnp.bool_)[..., None]
    return boxes, scores, labels, keep_mask


# speedup vs baseline: 12.3217x; 6.0537x over previous
import jax
import jax.numpy as jnp
from jax import lax
from jax.experimental import pallas as pl

_N = 20000
_HW = 256
_W = 16
_B = 8
_C = 2500


def _stage1_body(cls_ref, rois_ref, scores_ref, mask_ref, idx_ref,
                 base_ref, scale_ref):
    x = cls_ref[...]
    m = jnp.max(x, axis=0, keepdims=True)
    cell = lax.broadcasted_iota(jnp.int32, x.shape, 0)
    idx = jnp.min(jnp.where(x == m, cell, _HW), axis=0, keepdims=True)
    fw = (idx % _W).astype(jnp.float32)
    fh = (idx // _W).astype(jnp.float32)

    r = rois_ref[...]
    x1, y1, x2, y2 = r[0:1], r[1:2], r[2:3], r[3:4]
    cx = (x1 + x2) * 0.5
    cy = (y1 + y2) * 0.5
    hw = (x2 - x1) * 0.5
    hh = (y2 - y1) * 0.5
    nx1 = cx - hw
    ny1 = cy - hh
    bw = ((cx + hw) - nx1) * (1.0 / _W)
    bh = ((cy + hh) - ny1) * (1.0 / _W)

    scores_ref[...] = m
    mask_ref[...] = jnp.where(m >= 0.0, 1.0, 0.0)
    idx_ref[...] = idx
    bx = bw * (fw + 0.5) + nx1
    by = bh * (fh + 0.5) + ny1
    base_ref[...] = jnp.concatenate([bx, by, bx, by], axis=0)
    scale_ref[...] = jnp.concatenate([bw, bh, bw, bh], axis=0)


def _stage1(cls2, rois4):
    return pl.pallas_call(
        _stage1_body,
        in_specs=[
            pl.BlockSpec((_HW, _N), lambda: (0, 0)),
            pl.BlockSpec((4, _N), lambda: (0, 0)),
        ],
        out_specs=[
            pl.BlockSpec((1, _N), lambda: (0, 0)),
            pl.BlockSpec((1, _N), lambda: (0, 0)),
            pl.BlockSpec((1, _N), lambda: (0, 0)),
            pl.BlockSpec((4, _N), lambda: (0, 0)),
            pl.BlockSpec((4, _N), lambda: (0, 0)),
        ],
        out_shape=[
            jax.ShapeDtypeStruct((1, _N), jnp.float32),
            jax.ShapeDtypeStruct((1, _N), jnp.float32),
            jax.ShapeDtypeStruct((1, _N), jnp.int32),
            jax.ShapeDtypeStruct((4, _N), jnp.float32),
            jax.ShapeDtypeStruct((4, _N), jnp.float32),
        ],
    )(cls2, rois4)


def _stage2_body(reg_ref, idx_ref, base_ref, scale_ref, out_ref):
    g = pl.program_id(0)
    rg = reg_ref[...]
    idx = idx_ref[...]
    cell = lax.broadcasted_iota(jnp.int32, rg.shape, 0)
    off = jnp.sum(jnp.where(cell == idx, rg, 0.0), axis=0, keepdims=True)
    out_ref[pl.ds(g, 1)] = base_ref[pl.ds(g, 1)] + scale_ref[pl.ds(g, 1)] * off


def _stage2(reg2, idx2, base4, scale4):
    return pl.pallas_call(
        _stage2_body,
        grid=(4,),
        in_specs=[
            pl.BlockSpec((_HW, _N), lambda g: (g, 0)),
            pl.BlockSpec((1, _N), lambda g: (0, 0)),
            pl.BlockSpec((4, _N), lambda g: (0, 0)),
            pl.BlockSpec((4, _N), lambda g: (0, 0)),
        ],
        out_specs=pl.BlockSpec((4, _N), lambda g: (0, 0)),
        out_shape=jax.ShapeDtypeStruct((4, _N), jnp.float32),
    )(reg2, idx2, base4, scale4)


def kernel(batch_rois, rcnn_cls_pred, rcnn_reg_pred):
    cls2 = rcnn_cls_pred.reshape(_N, _HW).T
    reg2 = rcnn_reg_pred.reshape(_N, 4 * _HW).T
    rois4 = jnp.transpose(batch_rois, (2, 0, 1)).reshape(4, _N)

    scores2, mask2, idx2, base4, scale4 = _stage1(cls2, rois4)
    boxes4 = _stage2(reg2, idx2, base4, scale4)

    boxes = jnp.transpose(boxes4.reshape(4, _B, _C), (1, 2, 0))
    scores = scores2.reshape(_B, _C, 1)
    labels = jnp.zeros_like(scores)
    keep_mask = mask2.astype(jnp.bool_).reshape(_B, _C, 1)
    return boxes, scores, labels, keep_mask


# speedup vs baseline: 12.8806x; 1.0454x over previous
import jax
import jax.numpy as jnp
from jax import lax
from jax.experimental import pallas as pl
from jax.experimental.pallas import tpu as pltpu

_N = 20000
_HW = 256
_W = 16
_B = 8
_C = 2500
_RB = 128
_NCH = 4 * _HW // _RB


def _body(cls_ref, rois_ref, reg_ref,
          scores_ref, mask_ref, boxes_ref,
          idx_s, base_s, scale_s, acc_s):
    g = pl.program_id(0)

    @pl.when(g == 0)
    def _stage1():
        x = cls_ref[...]
        m = jnp.max(x, axis=0, keepdims=True)
        cell = lax.broadcasted_iota(jnp.int32, x.shape, 0)
        idx = jnp.min(jnp.where(x == m, cell, _HW), axis=0, keepdims=True)
        fw = (idx % _W).astype(jnp.float32)
        fh = (idx // _W).astype(jnp.float32)

        r = rois_ref[...]
        x1, y1, x2, y2 = r[0:1], r[1:2], r[2:3], r[3:4]
        cx = (x1 + x2) * 0.5
        cy = (y1 + y2) * 0.5
        hw = (x2 - x1) * 0.5
        hh = (y2 - y1) * 0.5
        nx1 = cx - hw
        ny1 = cy - hh
        bw = ((cx + hw) - nx1) * (1.0 / _W)
        bh = ((cy + hh) - ny1) * (1.0 / _W)

        scores_ref[...] = m
        mask_ref[...] = jnp.where(m >= 0.0, 1.0, 0.0)
        idx_s[...] = idx
        bx = bw * (fw + 0.5) + nx1
        by = bh * (fh + 0.5) + ny1
        base_s[...] = jnp.concatenate([bx, by, bx, by], axis=0)
        scale_s[...] = jnp.concatenate([bw, bh, bw, bh], axis=0)

    @pl.when(g > 0)
    def _stage2():
        c = g - 1
        k = c // 2
        half = c % 2
        rg = reg_ref[...]
        cell = lax.broadcasted_iota(jnp.int32, rg.shape, 0) + half * _RB
        part = jnp.sum(jnp.where(cell == idx_s[...], rg, 0.0),
                       axis=0, keepdims=True)

        @pl.when(half == 0)
        def _():
            acc_s[...] = part

        @pl.when(half == 1)
        def _():
            off = acc_s[...] + part
            boxes_ref[pl.ds(k, 1)] = (base_s[pl.ds(k, 1)]
                                      + scale_s[pl.ds(k, 1)] * off)


def _fused(cls2, rois4, reg2):
    return pl.pallas_call(
        _body,
        grid=(1 + _NCH,),
        in_specs=[
            pl.BlockSpec((_HW, _N), lambda g: (0, 0)),
            pl.BlockSpec((4, _N), lambda g: (0, 0)),
            pl.BlockSpec((_RB, _N), lambda g: (jnp.maximum(g - 1, 0), 0)),
        ],
        out_specs=[
            pl.BlockSpec((1, _N), lambda g: (0, 0)),
            pl.BlockSpec((1, _N), lambda g: (0, 0)),
            pl.BlockSpec((4, _N), lambda g: (0, 0)),
        ],
        out_shape=[
            jax.ShapeDtypeStruct((1, _N), jnp.float32),
            jax.ShapeDtypeStruct((1, _N), jnp.float32),
            jax.ShapeDtypeStruct((4, _N), jnp.float32),
        ],
        scratch_shapes=[
            pltpu.VMEM((1, _N), jnp.int32),
            pltpu.VMEM((4, _N), jnp.float32),
            pltpu.VMEM((4, _N), jnp.float32),
            pltpu.VMEM((1, _N), jnp.float32),
        ],
    )(cls2, rois4, reg2)


def kernel(batch_rois, rcnn_cls_pred, rcnn_reg_pred):
    cls2 = rcnn_cls_pred.reshape(_N, _HW).T
    reg2 = rcnn_reg_pred.reshape(_N, 4 * _HW).T
    rois4 = jnp.transpose(batch_rois, (2, 0, 1)).reshape(4, _N)

    scores2, mask2, boxes4 = _fused(cls2, rois4, reg2)

    boxes = jnp.transpose(boxes4.reshape(4, _B, _C), (1, 2, 0))
    scores = scores2.reshape(_B, _C, 1)
    labels = jnp.zeros_like(scores)
    keep_mask = mask2.astype(jnp.bool_).reshape(_B, _C, 1)
    return boxes, scores, labels, keep_mask
